# baseline (speedup 1.0000x reference)
import jax
import jax.numpy as jnp
from jax import lax
from jax.experimental import pallas as pl
from jax.experimental.pallas import tpu as pltpu
from jax.experimental.pallas import tpu_sc as plsc

_MAXPOS = 77
_D = 768
_B = 4096
_L = 77
_N = _B * _L
_NC = 2
_NS = 16
_NW = _NC * _NS
_PER_W = _N // _NW
_K = 16
_NBUF = 4
_NCHUNK = _PER_W // _K
_NGROUP = _NCHUNK // _NBUF
_PADPOS = _MAXPOS
_LANES = 16


def _sc_body(table_hbm, ids_hbm, pospad_hbm, out_hbm, idx_v, pospad_v, buf_v,
             sg0, sg1, sg2, sg3, ss0, ss1, ss2, ss3):
    sg = [sg0, sg1, sg2, sg3]
    ss = [ss0, ss1, ss2, ss3]
    wid = lax.axis_index("s") * _NC + lax.axis_index("c")
    base = wid * _PER_W
    pltpu.sync_copy(ids_hbm.at[pl.ds(base, _PER_W)], idx_v)
    pltpu.sync_copy(pospad_hbm, pospad_v)

    def start_gather(c, b):
        pltpu.async_copy(
            table_hbm.at[idx_v.at[pl.ds(c * _K, _K)]], buf_v.at[b], sg[b]
        )

    def wait_gather(b):
        pltpu.make_async_copy(
            table_hbm.at[idx_v.at[pl.ds(0, _K)]], buf_v.at[b], sg[b]
        ).wait()

    def start_store(c, b):
        pltpu.async_copy(buf_v.at[b], out_hbm.at[pl.ds(base + c * _K, _K)],
                         ss[b])

    def wait_store(c, b):
        pltpu.make_async_copy(
            buf_v.at[b], out_hbm.at[pl.ds(base + c * _K, _K)], ss[b]
        ).wait()

    for b in range(_NBUF - 2):
        start_gather(b, b)

    def group(gi, carry):
        for b in range(_NBUF):
            c = gi * _NBUF + b
            b2 = (b + 2) % _NBUF

            @pl.when(c >= 2)
            def _():
                wait_store(c - 2, b2)

            @pl.when(c + 2 < _NCHUNK)
            def _():
                start_gather(c + 2, b2)

            wait_gather(b)
            p0 = lax.rem(c * _K, _MAXPOS)

            @plsc.parallel_loop(0, _K, unroll=2)
            def _(i):
                p = p0 + i
                p = jnp.where(p >= _MAXPOS, p - _MAXPOS, p)
                for k in range(0, _D, _LANES):
                    plsc.addupdate(
                        buf_v.at[b, i, pl.ds(k, _LANES)],
                        pospad_v[p, pl.ds(k, _LANES)],
                    )

            start_store(c, b)
        return carry

    lax.fori_loop(0, _NGROUP, group, 0)
    wait_store(_NCHUNK - 2, (_NCHUNK - 2) % _NBUF)
    wait_store(_NCHUNK - 1, (_NCHUNK - 1) % _NBUF)


def kernel(input_ids, token_table, pos_table):
    ids_flat = input_ids.reshape(_N)

    mesh = plsc.VectorSubcoreMesh(core_axis_name="c", subcore_axis_name="s")
    run = pl.kernel(
        _sc_body,
        mesh=mesh,
        out_type=jax.ShapeDtypeStruct((_N, _D), jnp.float32),
        compiler_params=pltpu.CompilerParams(use_tc_tiling_on_sc=True),
        scratch_types=[
            pltpu.VMEM((_PER_W,), jnp.int32),
            pltpu.VMEM((_PADPOS, _D), jnp.float32),
            pltpu.VMEM((_NBUF, _K, _D), jnp.float32),
            pltpu.SemaphoreType.DMA,
            pltpu.SemaphoreType.DMA,
            pltpu.SemaphoreType.DMA,
            pltpu.SemaphoreType.DMA,
            pltpu.SemaphoreType.DMA,
            pltpu.SemaphoreType.DMA,
            pltpu.SemaphoreType.DMA,
            pltpu.SemaphoreType.DMA,
        ],
    )
    out = run(token_table, ids_flat, pos_table)
    return out.reshape(_B, _L, _D)


# speedup vs baseline: 1.4719x; 1.4719x over previous
import jax
import jax.numpy as jnp
from jax import lax
from jax.experimental import pallas as pl
from jax.experimental.pallas import tpu as pltpu
from jax.experimental.pallas import tpu_sc as plsc

_MAXPOS = 77
_D = 768
_B = 4096
_L = 77
_LP = 80
_NC = 2
_NS = 16
_NW = _NC * _NS
_SEQ_W = _B // _NW
_HALF = _SEQ_W // 2
_NBUF = 5
_K = 16
_LANES = 16


def _sc_body(table_hbm, ids_hbm, pos_hbm, out_hbm, idx_v, pos_v,
             buf0, buf1, buf2, buf3, buf4,
             sg0, sg1, sg2, sg3, sg4, ss0, ss1, ss2, ss3, ss4):
    bufs = [buf0, buf1, buf2, buf3, buf4]
    sg = [sg0, sg1, sg2, sg3, sg4]
    ss = [ss0, ss1, ss2, ss3, ss4]
    wid = lax.axis_index("s") * _NC + lax.axis_index("c")
    rt0 = wid * 0

    def start_gather(s, j):
        pltpu.async_copy(
            table_hbm.at[idx_v.at[pl.ds(s * _LP + j * _K, _K)]],
            bufs[j], sg[j],
        )

    def wait_gather(j):
        pltpu.make_async_copy(
            table_hbm.at[idx_v.at[pl.ds(0, _K)]], bufs[j], sg[j]
        ).wait()

    def start_store(b, j):
        pltpu.async_copy(
            bufs[j], out_hbm.at[b, pl.ds(rt0 + j * _K, _K), :], ss[j]
        )

    def wait_store(b, j):
        pltpu.make_async_copy(
            bufs[j], out_hbm.at[b, pl.ds(rt0 + j * _K, _K), :], ss[j]
        ).wait()

    pltpu.sync_copy(pos_hbm, pos_v)

    for h in range(2):
        seq0 = wid * _SEQ_W + h * _HALF
        pltpu.sync_copy(
            ids_hbm.at[pl.ds((wid * _SEQ_W + h * _HALF) * _LP, _HALF * _LP)],
            idx_v,
        )
        start_gather(0, 0)
        start_gather(0, 1)

        def seq_step(sl, carry):
            for j in range(_NBUF):
                c = sl * _NBUF + j
                j2 = (j + 2) % _NBUF
                s2 = sl + (j + 2) // _NBUF

                @pl.when(c >= 3)
                def _():
                    wait_store(seq0 + s2 - 1, j2)

                @pl.when(s2 < _HALF)
                def _():
                    start_gather(s2, j2)

                wait_gather(j)

                @plsc.parallel_loop(0, _K, unroll=2)
                def _(i):
                    for k in range(0, _D, _LANES):
                        plsc.addupdate(
                            bufs[j].at[i, pl.ds(k, _LANES)],
                            pos_v[j * _K + i, pl.ds(k, _LANES)],
                        )

                start_store(seq0 + sl, j)
            return carry

        lax.fori_loop(0, _HALF, seq_step, 0)
        for j in range(2, _NBUF):
            wait_store(seq0 + _HALF - 1, j)


def kernel(input_ids, token_table, pos_table):
    ids_pad = jnp.pad(input_ids, ((0, 0), (0, _LP - _L)),
                      mode="edge").reshape(_B * _LP)
    pos_pad = jnp.pad(pos_table, ((0, _LP - _MAXPOS), (0, 0)))

    mesh = plsc.VectorSubcoreMesh(core_axis_name="c", subcore_axis_name="s")
    run = pl.kernel(
        _sc_body,
        mesh=mesh,
        out_type=jax.ShapeDtypeStruct((_B, _L, _D), jnp.float32),
        compiler_params=pltpu.CompilerParams(use_tc_tiling_on_sc=True),
        scratch_types=[
            pltpu.VMEM((_HALF * _LP,), jnp.int32),
            pltpu.VMEM((_LP, _D), jnp.float32),
            pltpu.VMEM((_K, _D), jnp.float32),
            pltpu.VMEM((_K, _D), jnp.float32),
            pltpu.VMEM((_K, _D), jnp.float32),
            pltpu.VMEM((_K, _D), jnp.float32),
            pltpu.VMEM((_K, _D), jnp.float32),
            pltpu.SemaphoreType.DMA,
            pltpu.SemaphoreType.DMA,
            pltpu.SemaphoreType.DMA,
            pltpu.SemaphoreType.DMA,
            pltpu.SemaphoreType.DMA,
            pltpu.SemaphoreType.DMA,
            pltpu.SemaphoreType.DMA,
            pltpu.SemaphoreType.DMA,
            pltpu.SemaphoreType.DMA,
            pltpu.SemaphoreType.DMA,
        ],
    )
    return run(token_table, ids_pad, pos_pad)


# speedup vs baseline: 3.4846x; 2.3675x over previous
import jax
import jax.numpy as jnp
from jax import lax
from jax.experimental import pallas as pl
from jax.experimental.pallas import tpu as pltpu
from jax.experimental.pallas import tpu_sc as plsc

_MAXPOS = 77
_D = 768
_B = 4096
_L = 77
_NC = 2
_NS = 16
_NW = _NC * _NS
_SEQ_W = _B // _NW
_K = 16
_QW = _SEQ_W // _K
_NBUF = _QW
_PRE = 6
_NCHUNK = _L * _QW
_POSP = 16
_LANES = 16


def _sc_body(table_hbm, ids_hbm, pos_hbm, out_hbm, idx_v, pos_v,
             buf0, buf1, buf2, buf3, buf4, buf5, buf6, buf7,
             sg0, sg1, sg2, sg3, sg4, sg5, sg6, sg7,
             ss0, ss1, ss2, ss3, ss4, ss5, ss6, ss7):
    bufs = [buf0, buf1, buf2, buf3, buf4, buf5, buf6, buf7]
    sg = [sg0, sg1, sg2, sg3, sg4, sg5, sg6, sg7]
    ss = [ss0, ss1, ss2, ss3, ss4, ss5, ss6, ss7]
    wid = lax.axis_index("s") * _NC + lax.axis_index("c")
    b0 = wid * _SEQ_W

    def start_gather(c, j):
        pltpu.async_copy(
            table_hbm.at[idx_v.at[c // _QW, pl.ds((c % _QW) * _K, _K)]],
            bufs[j], sg[j],
        )

    def wait_gather(j):
        pltpu.make_async_copy(
            table_hbm.at[idx_v.at[0, pl.ds(0, _K)]], bufs[j], sg[j]
        ).wait()

    def start_store(c, j):
        pltpu.async_copy(
            bufs[j],
            out_hbm.at[c // _QW, pl.ds(b0 + (c % _QW) * _K, _K), :], ss[j],
        )

    def wait_store(c, j):
        pltpu.make_async_copy(
            bufs[j],
            out_hbm.at[c // _QW, pl.ds(b0 + (c % _QW) * _K, _K), :], ss[j],
        ).wait()

    pltpu.sync_copy(ids_hbm.at[:, pl.ds(b0, _SEQ_W)], idx_v)

    for c in range(_PRE):
        start_gather(c, c)

    def step(t, carry):
        @pl.when(t & (_POSP - 1) == 0)
        def _():
            off = pl.multiple_of(t & ~(_POSP - 1), _POSP)
            pltpu.sync_copy(pos_hbm.at[pl.ds(off, _POSP)], pos_v)

        for j in range(_NBUF):
            c = t * _NBUF + j
            j2 = (j + _PRE) % _NBUF

            @pl.when(c >= 2)
            def _():
                wait_store(c - 2, j2)

            @pl.when(c + _PRE < _NCHUNK)
            def _():
                start_gather(c + _PRE, j2)

            wait_gather(j)

            @plsc.parallel_loop(0, _D // _LANES, unroll=2)
            def _(k):
                v = pos_v[t & (_POSP - 1), pl.ds(k * _LANES, _LANES)]
                for i in range(_K):
                    plsc.addupdate(bufs[j].at[i, pl.ds(k * _LANES, _LANES)], v)

            start_store(c, j)
        return carry

    lax.fori_loop(0, _L, step, 0)
    wait_store(_NCHUNK - 2, (_NCHUNK - 2) % _NBUF)
    wait_store(_NCHUNK - 1, (_NCHUNK - 1) % _NBUF)


def kernel(input_ids, token_table, pos_table):
    ids_t = input_ids.T
    pos_pad = jnp.pad(pos_table, ((0, 80 - _MAXPOS), (0, 0)))

    mesh = plsc.VectorSubcoreMesh(core_axis_name="c", subcore_axis_name="s")
    run = pl.kernel(
        _sc_body,
        mesh=mesh,
        out_type=jax.ShapeDtypeStruct((_L, _B, _D), jnp.float32),
        compiler_params=pltpu.CompilerParams(use_tc_tiling_on_sc=True),
        scratch_types=(
            [
                pltpu.VMEM((_L, _SEQ_W), jnp.int32),
                pltpu.VMEM((_POSP, _D), jnp.float32),
            ]
            + [pltpu.VMEM((_K, _D), jnp.float32) for _ in range(_NBUF)]
            + [pltpu.SemaphoreType.DMA for _ in range(2 * _NBUF)]
        ),
    )
    out = run(token_table, ids_t, pos_pad)
    return out.transpose(1, 0, 2)
